# baseline (speedup 1.0000x reference)
import jax
import jax.numpy as jnp
from jax import lax
from jax.experimental import pallas as pl
from jax.experimental.pallas import tpu as pltpu
from jax.experimental.pallas import tpu_sc as plsc

N = 16 * 512 * 512
NC, NS = 2, 16
NW = NC * NS
PER_TILE = N // NW
K = 65536
KS = K // NS

C1 = 16384
C2 = 8192
R2 = C2 // 128

_mesh = plsc.VectorSubcoreMesh(core_axis_name="c", subcore_axis_name="s")


def _minmax_body(pred_hbm, label_hbm, mm_hbm, pbuf, lbuf, obuf):
    cid = lax.axis_index("c")
    sid = lax.axis_index("s")
    wid = cid * NS + sid
    base = wid * PER_TILE

    def chunk_body(ci, carry):
        lo, hi = carry
        off = base + ci * C1
        pltpu.sync_copy(pred_hbm.at[pl.ds(off, C1)], pbuf)
        pltpu.sync_copy(label_hbm.at[pl.ds(off, C1)], lbuf)

        def vec_body(i, c2):
            lo2, hi2 = c2
            p = pbuf[pl.ds(i * 16, 16)]
            l = lbuf[pl.ds(i * 16, 16)]
            e = 1.0 - p * (2.0 * l - 1.0)
            return jnp.minimum(lo2, e), jnp.maximum(hi2, e)

        return lax.fori_loop(0, C1 // 16, vec_body, (lo, hi))

    big = jnp.full((16,), 3.0e38, jnp.float32)
    lo, hi = lax.fori_loop(0, PER_TILE // C1, chunk_body, (big, -big))
    obuf[0, :] = lo
    obuf[1, :] = hi
    pltpu.sync_copy(obuf, mm_hbm.at[pl.ds(2 * wid, 2)])


_minmax_call = pl.kernel(
    _minmax_body,
    out_type=jax.ShapeDtypeStruct((2 * NW, 16), jnp.float32),
    mesh=_mesh,
    scratch_types=[
        pltpu.VMEM((C1,), jnp.float32),
        pltpu.VMEM((C1,), jnp.float32),
        pltpu.VMEM((2, 16), jnp.float32),
    ],
)


def _hist_body(pred_hbm, label_hbm, mm_hbm, hist_hbm,
               pbuf, lbuf, rbuf, idxbuf, onesbuf, mmbuf, hn, hp, hs):
    cid = lax.axis_index("c")
    sid = lax.axis_index("s")
    base = (cid * NS + sid) * PER_TILE

    pltpu.sync_copy(mm_hbm, mmbuf)
    lo = mmbuf[0, :]
    hi = mmbuf[1, :]
    for j in range(1, NW):
        lo = jnp.minimum(lo, mmbuf[2 * j, :])
        hi = jnp.maximum(hi, mmbuf[2 * j + 1, :])
    _dn = lax.GatherDimensionNumbers(
        offset_dims=(), collapsed_slice_dims=(0,), start_index_map=(0,))

    def _permute(x, perm):
        return lax.gather(x, perm[:, None], _dn, (1,),
                          mode=lax.GatherScatterMode.PROMISE_IN_BOUNDS)

    for mbit in (8, 4, 2, 1):
        perm = lax.iota(jnp.int32, 16) ^ mbit
        lo = jnp.minimum(lo, _permute(lo, perm))
        hi = jnp.maximum(hi, _permute(hi, perm))
    hi_v = hi
    iw_v = (K - 1.0) / jnp.maximum(hi - lo, 1e-30)

    zero16 = jnp.zeros((16,), jnp.float32)

    def z_body(i, _):
        rbuf[pl.ds(i * 16, 16)] = zero16
        return 0

    lax.fori_loop(0, KS // 16, z_body, 0)
    pltpu.sync_copy(rbuf.at[pl.ds(0, KS)], hn.at[pl.ds(sid * KS, KS)])
    pltpu.sync_copy(rbuf.at[pl.ds(0, KS)], hp.at[pl.ds(sid * KS, KS)])
    pltpu.sync_copy(rbuf.at[pl.ds(0, KS)], hs.at[pl.ds(sid * KS, KS)])

    one16 = jnp.ones((16,), jnp.float32)
    for i in range(8):
        onesbuf[pl.ds(i * 16, 16)] = one16

    plsc.subcore_barrier()

    def chunk(ci, _):
        off = base + ci * C2
        pltpu.sync_copy(pred_hbm.at[pl.ds(off, C2)], pbuf)
        pltpu.sync_copy(label_hbm.at[pl.ds(off, C2)], lbuf)

        def vec_body(i, _2):
            p = pbuf[pl.ds(i * 16, 16)]
            l = lbuf[pl.ds(i * 16, 16)]
            e = 1.0 - p * (2.0 * l - 1.0)
            r = jnp.maximum(e, 0.0)
            t = jnp.clip((hi_v - e) * iw_v, 0.0, K - 1.0)
            idx = t.astype(jnp.int32)
            rbuf[pl.ds(i * 16, 16)] = r
            row = i // 8
            col = (i % 8) * 16
            idxbuf[row, pl.ds(col, 16)] = idx
            return 0

        lax.fori_loop(0, C2 // 16, vec_body, 0)

        def row_body(j, _2):
            irow = idxbuf.at[j]
            pltpu.sync_copy(onesbuf, hn.at[irow], add=True)
            pltpu.sync_copy(lbuf.at[pl.ds(j * 128, 128)], hp.at[irow], add=True)
            pltpu.sync_copy(rbuf.at[pl.ds(j * 128, 128)], hs.at[irow], add=True)
            return 0

        lax.fori_loop(0, R2, row_body, 0)
        return 0

    lax.fori_loop(0, PER_TILE // C2, chunk, 0)
    plsc.subcore_barrier()

    hbase = cid * (3 * K)
    pltpu.sync_copy(hn.at[pl.ds(sid * KS, KS)],
                    hist_hbm.at[pl.ds(hbase + sid * KS, KS)])
    pltpu.sync_copy(hp.at[pl.ds(sid * KS, KS)],
                    hist_hbm.at[pl.ds(hbase + K + sid * KS, KS)])
    pltpu.sync_copy(hs.at[pl.ds(sid * KS, KS)],
                    hist_hbm.at[pl.ds(hbase + 2 * K + sid * KS, KS)])


_hist_call = pl.kernel(
    _hist_body,
    out_type=jax.ShapeDtypeStruct((NC * 3 * K,), jnp.float32),
    mesh=_mesh,
    scratch_types=[
        pltpu.VMEM((C2,), jnp.float32),
        pltpu.VMEM((C2,), jnp.float32),
        pltpu.VMEM((C2,), jnp.float32),
        pltpu.VMEM((R2, 128), jnp.int32),
        pltpu.VMEM((128,), jnp.float32),
        pltpu.VMEM((2 * NW, 16), jnp.float32),
        pltpu.VMEM_SHARED((K,), jnp.float32),
        pltpu.VMEM_SHARED((K,), jnp.float32),
        pltpu.VMEM_SHARED((K,), jnp.float32),
    ],
)


def _finish_body(h_ref, o_ref):
    n = h_ref[0, 0] + h_ref[1, 0]
    p = h_ref[0, 1] + h_ref[1, 1]
    s = h_ref[0, 2] + h_ref[1, 2]
    m = n - p
    rows = lax.broadcasted_iota(jnp.int32, (256, 256), 0)
    cols = lax.broadcasted_iota(jnp.int32, (256, 256), 1)
    ut = (rows <= cols).astype(jnp.float32)
    slt = (cols < rows).astype(jnp.float32)
    cp = jnp.dot(p, ut, preferred_element_type=jnp.float32,
                 precision=lax.Precision.HIGHEST)
    cn = jnp.dot(m, ut, preferred_element_type=jnp.float32,
                 precision=lax.Precision.HIGHEST)
    prow = jnp.sum(p, axis=1, keepdims=True)
    mrow = jnp.sum(m, axis=1, keepdims=True)
    cp = cp + jnp.dot(slt, prow, preferred_element_type=jnp.float32,
                 precision=lax.Precision.HIGHEST)
    cn = cn + jnp.dot(slt, mrow, preferred_element_type=jnp.float32,
                 precision=lax.Precision.HIGHEST)
    big_p = jnp.sum(p)
    cnt = cp + cn
    iou = jnp.where(cnt > 0.5,
                    1.0 - (big_p - cp) / jnp.maximum(big_p + cn, 1.0),
                    0.0)
    shift = (rows == cols - 1).astype(jnp.float32)
    prev_in_row = jnp.dot(iou, shift, preferred_element_type=jnp.float32,
                 precision=lax.Precision.HIGHEST)
    lastcol = iou[:, 255:256]
    shiftr = (cols == rows - 1).astype(jnp.float32)
    prev_row_last = jnp.dot(shiftr, lastcol, preferred_element_type=jnp.float32,
                 precision=lax.Precision.HIGHEST)
    col0 = (cols == 0).astype(jnp.float32)
    prev = prev_in_row + col0 * prev_row_last
    d = iou - prev
    loss = jnp.sum(s / jnp.maximum(n, 1.0) * d)
    o_ref[...] = loss.reshape(1, 1)


_finish_call = pl.pallas_call(
    _finish_body,
    out_shape=jax.ShapeDtypeStruct((1, 1), jnp.float32),
    in_specs=[pl.BlockSpec((2, 3, 256, 256), lambda: (0, 0, 0, 0))],
    out_specs=pl.BlockSpec((1, 1), lambda: (0, 0)),
)


def kernel(prediction, label):
    pred = prediction.reshape(-1)
    lab = label.reshape(-1)
    mm = _minmax_call(pred, lab)
    hist = _hist_call(pred, lab, mm)
    out = _finish_call(hist.reshape(2, 3, 256, 256))
    return out.reshape(())


# speedup vs baseline: 1.7066x; 1.7066x over previous
import jax
import jax.numpy as jnp
from jax import lax
from jax.experimental import pallas as pl
from jax.experimental.pallas import tpu as pltpu
from jax.experimental.pallas import tpu_sc as plsc

N = 16 * 512 * 512
NC, NS = 2, 16
NW = NC * NS
PER_TILE = N // NW
K = 65536
KS = K // NS

C1 = 16384
C2 = 8192
R2 = C2 // 128

_mesh = plsc.VectorSubcoreMesh(core_axis_name="c", subcore_axis_name="s")


def _minmax_body(pred_hbm, label_hbm, mm_hbm, pbuf, lbuf, obuf):
    cid = lax.axis_index("c")
    sid = lax.axis_index("s")
    wid = cid * NS + sid
    base = wid * PER_TILE

    def chunk_body(ci, carry):
        lo, hi = carry
        off = base + ci * C1
        pltpu.sync_copy(pred_hbm.at[pl.ds(off, C1)], pbuf)
        pltpu.sync_copy(label_hbm.at[pl.ds(off, C1)], lbuf)

        def vec_body(i, c2):
            lo2, hi2 = c2
            p = pbuf[pl.ds(i * 16, 16)]
            l = lbuf[pl.ds(i * 16, 16)]
            e = 1.0 - p * (2.0 * l - 1.0)
            return jnp.minimum(lo2, e), jnp.maximum(hi2, e)

        return lax.fori_loop(0, C1 // 16, vec_body, (lo, hi))

    big = jnp.full((16,), 3.0e38, jnp.float32)
    lo, hi = lax.fori_loop(0, PER_TILE // C1, chunk_body, (big, -big))
    obuf[0, :] = lo
    obuf[1, :] = hi
    pltpu.sync_copy(obuf, mm_hbm.at[pl.ds(2 * wid, 2)])


_minmax_call = pl.kernel(
    _minmax_body,
    out_type=jax.ShapeDtypeStruct((2 * NW, 16), jnp.float32),
    mesh=_mesh,
    scratch_types=[
        pltpu.VMEM((C1,), jnp.float32),
        pltpu.VMEM((C1,), jnp.float32),
        pltpu.VMEM((2, 16), jnp.float32),
    ],
)


def _hist_body(pred_hbm, label_hbm, mm_hbm, hist_hbm,
               pbuf, lbuf, rbuf, idxbuf, onesbuf, mmbuf, hn, hp, hs,
               sem_in, sem_sc):
    cid = lax.axis_index("c")
    sid = lax.axis_index("s")
    base = (cid * NS + sid) * PER_TILE

    pltpu.sync_copy(mm_hbm, mmbuf)
    lo = mmbuf[0, :]
    hi = mmbuf[1, :]
    for j in range(1, NW):
        lo = jnp.minimum(lo, mmbuf[2 * j, :])
        hi = jnp.maximum(hi, mmbuf[2 * j + 1, :])
    _dn = lax.GatherDimensionNumbers(
        offset_dims=(), collapsed_slice_dims=(0,), start_index_map=(0,))

    def _permute(x, perm):
        return lax.gather(x, perm[:, None], _dn, (1,),
                          mode=lax.GatherScatterMode.PROMISE_IN_BOUNDS)

    for mbit in (8, 4, 2, 1):
        perm = lax.iota(jnp.int32, 16) ^ mbit
        lo = jnp.minimum(lo, _permute(lo, perm))
        hi = jnp.maximum(hi, _permute(hi, perm))
    hi_v = hi
    iw_v = (K - 1.0) / jnp.maximum(hi - lo, 1e-30)

    zero16 = jnp.zeros((16,), jnp.float32)

    def z_body(i, _):
        rbuf[pl.ds(i * 16, 16)] = zero16
        return 0

    lax.fori_loop(0, KS // 16, z_body, 0)
    pltpu.sync_copy(rbuf.at[pl.ds(0, KS)], hn.at[pl.ds(sid * KS, KS)])
    pltpu.sync_copy(rbuf.at[pl.ds(0, KS)], hp.at[pl.ds(sid * KS, KS)])
    pltpu.sync_copy(rbuf.at[pl.ds(0, KS)], hs.at[pl.ds(sid * KS, KS)])

    one16 = jnp.ones((16,), jnp.float32)
    for i in range(8):
        onesbuf[pl.ds(i * 16, 16)] = one16

    NCHUNK = PER_TILE // C2

    pltpu.async_copy(pred_hbm.at[pl.ds(base, C2)],
                     pbuf.at[pl.ds(0, C2)], sem_in)
    pltpu.async_copy(label_hbm.at[pl.ds(base, C2)],
                     lbuf.at[pl.ds(0, C2)], sem_in)

    plsc.subcore_barrier()

    def chunk(ci, _):
        b = lax.rem(ci, 3)
        boff = b * C2
        pltpu.make_async_copy(pred_hbm.at[pl.ds(0, C2)],
                              pbuf.at[pl.ds(boff, C2)], sem_in).wait()
        pltpu.make_async_copy(pred_hbm.at[pl.ds(0, C2)],
                              lbuf.at[pl.ds(boff, C2)], sem_in).wait()

        @pl.when(ci < NCHUNK - 1)
        def _():
            b2 = lax.rem(ci + 1, 3)
            off2 = base + (ci + 1) * C2
            pltpu.async_copy(pred_hbm.at[pl.ds(off2, C2)],
                             pbuf.at[pl.ds(b2 * C2, C2)], sem_in)
            pltpu.async_copy(label_hbm.at[pl.ds(off2, C2)],
                             lbuf.at[pl.ds(b2 * C2, C2)], sem_in)

        def vec_body(i, _2):
            p = pbuf[pl.ds(boff + i * 16, 16)]
            l = lbuf[pl.ds(boff + i * 16, 16)]
            e = 1.0 - p * (2.0 * l - 1.0)
            r = jnp.maximum(e, 0.0)
            t = jnp.clip((hi_v - e) * iw_v, 0.0, K - 1.0)
            idx = t.astype(jnp.int32)
            rbuf[pl.ds(boff + i * 16, 16)] = r
            idxbuf[b * R2 + i // 8, pl.ds((i % 8) * 16, 16)] = idx
            return 0

        lax.fori_loop(0, C2 // 16, vec_body, 0)

        @pl.when(ci > 0)
        def _():
            pltpu.make_async_copy(pred_hbm.at[pl.ds(0, 3 * C2)],
                                  rbuf, sem_sc).wait()

        def row_body(j, _2):
            irow = idxbuf.at[b * R2 + j]
            src_off = boff + j * 128
            pltpu.async_copy(onesbuf, hn.at[irow], sem_sc, add=True)
            pltpu.async_copy(lbuf.at[pl.ds(src_off, 128)],
                             hp.at[irow], sem_sc, add=True)
            pltpu.async_copy(rbuf.at[pl.ds(src_off, 128)],
                             hs.at[irow], sem_sc, add=True)
            return 0

        lax.fori_loop(0, R2, row_body, 0)
        return 0

    lax.fori_loop(0, NCHUNK, chunk, 0)
    pltpu.make_async_copy(pred_hbm.at[pl.ds(0, 3 * C2)], rbuf, sem_sc).wait()
    plsc.subcore_barrier()

    hbase = cid * (3 * K)
    pltpu.sync_copy(hn.at[pl.ds(sid * KS, KS)],
                    hist_hbm.at[pl.ds(hbase + sid * KS, KS)])
    pltpu.sync_copy(hp.at[pl.ds(sid * KS, KS)],
                    hist_hbm.at[pl.ds(hbase + K + sid * KS, KS)])
    pltpu.sync_copy(hs.at[pl.ds(sid * KS, KS)],
                    hist_hbm.at[pl.ds(hbase + 2 * K + sid * KS, KS)])


_hist_call = pl.kernel(
    _hist_body,
    out_type=jax.ShapeDtypeStruct((NC * 3 * K,), jnp.float32),
    mesh=_mesh,
    scratch_types=[
        pltpu.VMEM((3 * C2,), jnp.float32),
        pltpu.VMEM((3 * C2,), jnp.float32),
        pltpu.VMEM((3 * C2,), jnp.float32),
        pltpu.VMEM((3 * R2, 128), jnp.int32),
        pltpu.VMEM((128,), jnp.float32),
        pltpu.VMEM((2 * NW, 16), jnp.float32),
        pltpu.VMEM_SHARED((K,), jnp.float32),
        pltpu.VMEM_SHARED((K,), jnp.float32),
        pltpu.VMEM_SHARED((K,), jnp.float32),
        pltpu.SemaphoreType.DMA,
        pltpu.SemaphoreType.DMA,
    ],
)


def _finish_body(h_ref, o_ref):
    n = h_ref[0, 0] + h_ref[1, 0]
    p = h_ref[0, 1] + h_ref[1, 1]
    s = h_ref[0, 2] + h_ref[1, 2]
    m = n - p
    rows = lax.broadcasted_iota(jnp.int32, (256, 256), 0)
    cols = lax.broadcasted_iota(jnp.int32, (256, 256), 1)
    ut = (rows <= cols).astype(jnp.float32)
    slt = (cols < rows).astype(jnp.float32)
    cp = jnp.dot(p, ut, preferred_element_type=jnp.float32,
                 precision=lax.Precision.HIGHEST)
    cn = jnp.dot(m, ut, preferred_element_type=jnp.float32,
                 precision=lax.Precision.HIGHEST)
    prow = jnp.sum(p, axis=1, keepdims=True)
    mrow = jnp.sum(m, axis=1, keepdims=True)
    cp = cp + jnp.dot(slt, prow, preferred_element_type=jnp.float32,
                 precision=lax.Precision.HIGHEST)
    cn = cn + jnp.dot(slt, mrow, preferred_element_type=jnp.float32,
                 precision=lax.Precision.HIGHEST)
    big_p = jnp.sum(p)
    cnt = cp + cn
    iou = jnp.where(cnt > 0.5,
                    1.0 - (big_p - cp) / jnp.maximum(big_p + cn, 1.0),
                    0.0)
    shift = (rows == cols - 1).astype(jnp.float32)
    prev_in_row = jnp.dot(iou, shift, preferred_element_type=jnp.float32,
                 precision=lax.Precision.HIGHEST)
    lastcol = iou[:, 255:256]
    shiftr = (cols == rows - 1).astype(jnp.float32)
    prev_row_last = jnp.dot(shiftr, lastcol, preferred_element_type=jnp.float32,
                 precision=lax.Precision.HIGHEST)
    col0 = (cols == 0).astype(jnp.float32)
    prev = prev_in_row + col0 * prev_row_last
    d = iou - prev
    loss = jnp.sum(s / jnp.maximum(n, 1.0) * d)
    o_ref[...] = loss.reshape(1, 1)


_finish_call = pl.pallas_call(
    _finish_body,
    out_shape=jax.ShapeDtypeStruct((1, 1), jnp.float32),
    in_specs=[pl.BlockSpec((2, 3, 256, 256), lambda: (0, 0, 0, 0))],
    out_specs=pl.BlockSpec((1, 1), lambda: (0, 0)),
)


def kernel(prediction, label):
    pred = prediction.reshape(-1)
    lab = label.reshape(-1)
    mm = _minmax_call(pred, lab)
    hist = _hist_call(pred, lab, mm)
    out = _finish_call(hist.reshape(2, 3, 256, 256))
    return out.reshape(())


# speedup vs baseline: 2.1985x; 1.2882x over previous
import jax
import jax.numpy as jnp
from jax import lax
from jax.experimental import pallas as pl
from jax.experimental.pallas import tpu as pltpu
from jax.experimental.pallas import tpu_sc as plsc

N = 16 * 512 * 512
NC, NS = 2, 16
NW = NC * NS
PER_TILE = N // NW
K = 65536
KS = K // NS

C1 = 16384
C2 = 8192
R2 = C2 // 128

_mesh = plsc.VectorSubcoreMesh(core_axis_name="c", subcore_axis_name="s")


def _minmax_body(pred_hbm, label_hbm, mm_hbm, pbuf, lbuf, obuf, sem_in):
    cid = lax.axis_index("c")
    sid = lax.axis_index("s")
    wid = cid * NS + sid
    base = wid * PER_TILE
    NCHUNK = PER_TILE // C1

    pltpu.async_copy(pred_hbm.at[pl.ds(base, C1)],
                     pbuf.at[pl.ds(0, C1)], sem_in)
    pltpu.async_copy(label_hbm.at[pl.ds(base, C1)],
                     lbuf.at[pl.ds(0, C1)], sem_in)

    def chunk_body(ci, carry):
        lo, hi = carry
        b = lax.rem(ci, 2)
        boff = b * C1
        pltpu.make_async_copy(pred_hbm.at[pl.ds(0, C1)],
                              pbuf.at[pl.ds(boff, C1)], sem_in).wait()
        pltpu.make_async_copy(pred_hbm.at[pl.ds(0, C1)],
                              lbuf.at[pl.ds(boff, C1)], sem_in).wait()

        @pl.when(ci < NCHUNK - 1)
        def _():
            b2 = lax.rem(ci + 1, 2)
            off2 = base + (ci + 1) * C1
            pltpu.async_copy(pred_hbm.at[pl.ds(off2, C1)],
                             pbuf.at[pl.ds(b2 * C1, C1)], sem_in)
            pltpu.async_copy(label_hbm.at[pl.ds(off2, C1)],
                             lbuf.at[pl.ds(b2 * C1, C1)], sem_in)

        def vec_body(i, c2):
            lo2, hi2 = c2
            for u in range(8):
                off = boff + i * 128 + u * 16
                p = pbuf[pl.ds(off, 16)]
                l = lbuf[pl.ds(off, 16)]
                e = 1.0 - p * (2.0 * l - 1.0)
                lo2 = jnp.minimum(lo2, e)
                hi2 = jnp.maximum(hi2, e)
            return lo2, hi2

        return lax.fori_loop(0, C1 // 128, vec_body, (lo, hi))

    big = jnp.full((16,), 3.0e38, jnp.float32)
    lo, hi = lax.fori_loop(0, NCHUNK, chunk_body, (big, -big))
    obuf[0, :] = lo
    obuf[1, :] = hi
    pltpu.sync_copy(obuf, mm_hbm.at[pl.ds(2 * wid, 2)])


_minmax_call = pl.kernel(
    _minmax_body,
    out_type=jax.ShapeDtypeStruct((2 * NW, 16), jnp.float32),
    mesh=_mesh,
    scratch_types=[
        pltpu.VMEM((2 * C1,), jnp.float32),
        pltpu.VMEM((2 * C1,), jnp.float32),
        pltpu.VMEM((2, 16), jnp.float32),
        pltpu.SemaphoreType.DMA,
    ],
)


def _hist_body(pred_hbm, label_hbm, mm_hbm, hist_hbm,
               pbuf, lbuf, zbuf, idxbuf, onesbuf, mmbuf, hn, hp,
               sem_in, sem_sc):
    cid = lax.axis_index("c")
    sid = lax.axis_index("s")
    base = (cid * NS + sid) * PER_TILE

    pltpu.sync_copy(mm_hbm, mmbuf)
    lo = mmbuf[0, :]
    hi = mmbuf[1, :]
    for j in range(1, NW):
        lo = jnp.minimum(lo, mmbuf[2 * j, :])
        hi = jnp.maximum(hi, mmbuf[2 * j + 1, :])
    _dn = lax.GatherDimensionNumbers(
        offset_dims=(), collapsed_slice_dims=(0,), start_index_map=(0,))

    def _permute(x, perm):
        return lax.gather(x, perm[:, None], _dn, (1,),
                          mode=lax.GatherScatterMode.PROMISE_IN_BOUNDS)

    for mbit in (8, 4, 2, 1):
        perm = lax.iota(jnp.int32, 16) ^ mbit
        lo = jnp.minimum(lo, _permute(lo, perm))
        hi = jnp.maximum(hi, _permute(hi, perm))
    hi_v = hi
    iw_v = (K - 1.0) / jnp.maximum(hi - lo, 1e-30)

    zero16 = jnp.zeros((16,), jnp.float32)

    def z_body(i, _):
        zbuf[pl.ds(i * 16, 16)] = zero16
        return 0

    lax.fori_loop(0, KS // 16, z_body, 0)
    pltpu.sync_copy(zbuf.at[pl.ds(0, KS)], hn.at[pl.ds(sid * KS, KS)])
    pltpu.sync_copy(zbuf.at[pl.ds(0, KS)], hp.at[pl.ds(sid * KS, KS)])

    one16 = jnp.ones((16,), jnp.float32)
    for i in range(8):
        onesbuf[pl.ds(i * 16, 16)] = one16

    NCHUNK = PER_TILE // C2

    pltpu.async_copy(pred_hbm.at[pl.ds(base, C2)],
                     pbuf.at[pl.ds(0, C2)], sem_in)
    pltpu.async_copy(label_hbm.at[pl.ds(base, C2)],
                     lbuf.at[pl.ds(0, C2)], sem_in)

    plsc.subcore_barrier()

    def chunk(ci, _):
        b = lax.rem(ci, 3)
        boff = b * C2
        pltpu.make_async_copy(pred_hbm.at[pl.ds(0, C2)],
                              pbuf.at[pl.ds(boff, C2)], sem_in).wait()
        pltpu.make_async_copy(pred_hbm.at[pl.ds(0, C2)],
                              lbuf.at[pl.ds(boff, C2)], sem_in).wait()

        @pl.when(ci < NCHUNK - 1)
        def _():
            b2 = lax.rem(ci + 1, 3)
            off2 = base + (ci + 1) * C2
            pltpu.async_copy(pred_hbm.at[pl.ds(off2, C2)],
                             pbuf.at[pl.ds(b2 * C2, C2)], sem_in)
            pltpu.async_copy(label_hbm.at[pl.ds(off2, C2)],
                             lbuf.at[pl.ds(b2 * C2, C2)], sem_in)

        def vec_body(i, _2):
            for u in range(8):
                off = boff + i * 128 + u * 16
                p = pbuf[pl.ds(off, 16)]
                l = lbuf[pl.ds(off, 16)]
                e = 1.0 - p * (2.0 * l - 1.0)
                t = jnp.clip((hi_v - e) * iw_v, 0.0, K - 1.0)
                idxbuf[b * R2 + i, pl.ds(u * 16, 16)] = t.astype(jnp.int32)
            return 0

        lax.fori_loop(0, C2 // 128, vec_body, 0)

        @pl.when(ci > 0)
        def _():
            pltpu.make_async_copy(pred_hbm.at[pl.ds(0, 2 * C2)],
                                  zbuf, sem_sc).wait()

        def row_body(j, _2):
            irow = idxbuf.at[b * R2 + j]
            pltpu.async_copy(onesbuf, hn.at[irow], sem_sc, add=True)
            pltpu.async_copy(lbuf.at[pl.ds(boff + j * 128, 128)],
                             hp.at[irow], sem_sc, add=True)
            return 0

        lax.fori_loop(0, R2, row_body, 0)
        return 0

    lax.fori_loop(0, NCHUNK, chunk, 0)
    pltpu.make_async_copy(pred_hbm.at[pl.ds(0, 2 * C2)], zbuf, sem_sc).wait()
    plsc.subcore_barrier()

    hbase = cid * (2 * K)
    pltpu.sync_copy(hn.at[pl.ds(sid * KS, KS)],
                    hist_hbm.at[pl.ds(hbase + sid * KS, KS)])
    pltpu.sync_copy(hp.at[pl.ds(sid * KS, KS)],
                    hist_hbm.at[pl.ds(hbase + K + sid * KS, KS)])


_hist_call = pl.kernel(
    _hist_body,
    out_type=jax.ShapeDtypeStruct((NC * 2 * K,), jnp.float32),
    mesh=_mesh,
    scratch_types=[
        pltpu.VMEM((3 * C2,), jnp.float32),
        pltpu.VMEM((3 * C2,), jnp.float32),
        pltpu.VMEM((2 * C2,), jnp.float32),
        pltpu.VMEM((3 * R2, 128), jnp.int32),
        pltpu.VMEM((128,), jnp.float32),
        pltpu.VMEM((2 * NW, 16), jnp.float32),
        pltpu.VMEM_SHARED((K,), jnp.float32),
        pltpu.VMEM_SHARED((K,), jnp.float32),
        pltpu.SemaphoreType.DMA,
        pltpu.SemaphoreType.DMA,
    ],
)


def _finish_body(mm_ref, h_ref, o_ref):
    n = h_ref[0, 0] + h_ref[1, 0]
    p = h_ref[0, 1] + h_ref[1, 1]
    m = n - p
    lo = jnp.min(mm_ref[:, 0:16])
    hi = jnp.max(mm_ref[:, 16:32])
    rows = lax.broadcasted_iota(jnp.int32, (256, 256), 0)
    cols = lax.broadcasted_iota(jnp.int32, (256, 256), 1)
    ut = (rows <= cols).astype(jnp.float32)
    slt = (cols < rows).astype(jnp.float32)
    cp = jnp.dot(p, ut, preferred_element_type=jnp.float32,
                 precision=lax.Precision.HIGHEST)
    cn = jnp.dot(m, ut, preferred_element_type=jnp.float32,
                 precision=lax.Precision.HIGHEST)
    prow = jnp.sum(p, axis=1, keepdims=True)
    mrow = jnp.sum(m, axis=1, keepdims=True)
    cp = cp + jnp.dot(slt, prow, preferred_element_type=jnp.float32,
                      precision=lax.Precision.HIGHEST)
    cn = cn + jnp.dot(slt, mrow, preferred_element_type=jnp.float32,
                      precision=lax.Precision.HIGHEST)
    big_p = jnp.sum(p)
    cnt = cp + cn
    iou = jnp.where(cnt > 0.5,
                    1.0 - (big_p - cp) / jnp.maximum(big_p + cn, 1.0),
                    0.0)
    shift = (rows == cols - 1).astype(jnp.float32)
    prev_in_row = jnp.dot(iou, shift, preferred_element_type=jnp.float32,
                          precision=lax.Precision.HIGHEST)
    lastcol = iou[:, 255:256]
    shiftr = (cols == rows - 1).astype(jnp.float32)
    prev_row_last = jnp.dot(shiftr, lastcol, preferred_element_type=jnp.float32,
                            precision=lax.Precision.HIGHEST)
    col0 = (cols == 0).astype(jnp.float32)
    prev = prev_in_row + col0 * prev_row_last
    d = iou - prev
    w = jnp.maximum(hi - lo, 1e-30) / (K - 1.0)
    kk = (rows * 256 + cols).astype(jnp.float32)
    mid = jnp.maximum(hi - (kk + 0.5) * w, 0.0)
    loss = jnp.sum(mid * d)
    o_ref[...] = loss.reshape(1, 1)


_finish_call = pl.pallas_call(
    _finish_body,
    out_shape=jax.ShapeDtypeStruct((1, 1), jnp.float32),
    in_specs=[
        pl.BlockSpec((32, 32), lambda: (0, 0)),
        pl.BlockSpec((2, 2, 256, 256), lambda: (0, 0, 0, 0)),
    ],
    out_specs=pl.BlockSpec((1, 1), lambda: (0, 0)),
)


def kernel(prediction, label):
    pred = prediction.reshape(-1)
    lab = label.reshape(-1)
    mm = _minmax_call(pred, lab)
    hist = _hist_call(pred, lab, mm)
    out = _finish_call(mm.reshape(32, 32), hist.reshape(2, 2, 256, 256))
    return out.reshape(())


# speedup vs baseline: 2.4659x; 1.1216x over previous
import jax
import jax.numpy as jnp
from jax import lax
from jax.experimental import pallas as pl
from jax.experimental.pallas import tpu as pltpu
from jax.experimental.pallas import tpu_sc as plsc

N = 16 * 512 * 512
NC, NS = 2, 16
NW = NC * NS
PER_TILE = N // NW
K = 65536
KS = K // NS

C1 = 16384
C2 = 8192
R2 = C2 // 128

_mesh = plsc.VectorSubcoreMesh(core_axis_name="c", subcore_axis_name="s")


def _minmax_body(pred_hbm, label_hbm, mm_hbm, pbuf, lbuf, obuf, sem_in):
    cid = lax.axis_index("c")
    sid = lax.axis_index("s")
    wid = cid * NS + sid
    base = wid * PER_TILE
    NCHUNK = PER_TILE // C1

    pltpu.async_copy(pred_hbm.at[pl.ds(base, C1)],
                     pbuf.at[pl.ds(0, C1)], sem_in)
    pltpu.async_copy(label_hbm.at[pl.ds(base, C1)],
                     lbuf.at[pl.ds(0, C1)], sem_in)

    def chunk_body(ci, carry):
        lo, hi = carry
        b = lax.rem(ci, 2)
        boff = b * C1
        pltpu.make_async_copy(pred_hbm.at[pl.ds(0, C1)],
                              pbuf.at[pl.ds(boff, C1)], sem_in).wait()
        pltpu.make_async_copy(pred_hbm.at[pl.ds(0, C1)],
                              lbuf.at[pl.ds(boff, C1)], sem_in).wait()

        @pl.when(ci < NCHUNK - 1)
        def _():
            b2 = lax.rem(ci + 1, 2)
            off2 = base + (ci + 1) * C1
            pltpu.async_copy(pred_hbm.at[pl.ds(off2, C1)],
                             pbuf.at[pl.ds(b2 * C1, C1)], sem_in)
            pltpu.async_copy(label_hbm.at[pl.ds(off2, C1)],
                             lbuf.at[pl.ds(b2 * C1, C1)], sem_in)

        def vec_body(i, c2):
            lo2, hi2 = c2
            for u in range(8):
                off = boff + i * 128 + u * 16
                p = pbuf[pl.ds(off, 16)]
                l = lbuf[pl.ds(off, 16)]
                e = 1.0 - p * (2.0 * l - 1.0)
                lo2 = jnp.minimum(lo2, e)
                hi2 = jnp.maximum(hi2, e)
            return lo2, hi2

        return lax.fori_loop(0, C1 // 128, vec_body, (lo, hi))

    big = jnp.full((16,), 3.0e38, jnp.float32)
    lo, hi = lax.fori_loop(0, NCHUNK, chunk_body, (big, -big))
    obuf[0, :] = lo
    obuf[1, :] = hi
    pltpu.sync_copy(obuf, mm_hbm.at[pl.ds(2 * wid, 2)])


_minmax_call = pl.kernel(
    _minmax_body,
    out_type=jax.ShapeDtypeStruct((2 * NW, 16), jnp.float32),
    mesh=_mesh,
    scratch_types=[
        pltpu.VMEM((2 * C1,), jnp.float32),
        pltpu.VMEM((2 * C1,), jnp.float32),
        pltpu.VMEM((2, 16), jnp.float32),
        pltpu.SemaphoreType.DMA,
    ],
)


def _hist_body(pred_hbm, label_hbm, mm_hbm, hist_hbm,
               pbuf, lbuf, zbuf, idxbuf, onesbuf, mmbuf, h2,
               sem_in, sem_sc):
    cid = lax.axis_index("c")
    sid = lax.axis_index("s")
    base = (cid * NS + sid) * PER_TILE

    pltpu.sync_copy(mm_hbm, mmbuf)
    lo = mmbuf[0, :]
    hi = mmbuf[1, :]
    for j in range(1, NW):
        lo = jnp.minimum(lo, mmbuf[2 * j, :])
        hi = jnp.maximum(hi, mmbuf[2 * j + 1, :])
    _dn = lax.GatherDimensionNumbers(
        offset_dims=(), collapsed_slice_dims=(0,), start_index_map=(0,))

    def _permute(x, perm):
        return lax.gather(x, perm[:, None], _dn, (1,),
                          mode=lax.GatherScatterMode.PROMISE_IN_BOUNDS)

    for mbit in (8, 4, 2, 1):
        perm = lax.iota(jnp.int32, 16) ^ mbit
        lo = jnp.minimum(lo, _permute(lo, perm))
        hi = jnp.maximum(hi, _permute(hi, perm))
    hi_v = hi
    iw_v = (K - 1.0) / jnp.maximum(hi - lo, 1e-30)

    zero16 = jnp.zeros((16,), jnp.float32)

    def z_body(i, _):
        zbuf[pl.ds(i * 16, 16)] = zero16
        return 0

    lax.fori_loop(0, (2 * KS) // 16, z_body, 0)
    pltpu.sync_copy(zbuf, h2.at[pl.ds(sid * 2 * KS, 2 * KS)])

    one16 = jnp.ones((16,), jnp.float32)
    for i in range(8):
        onesbuf[pl.ds(i * 16, 16)] = one16

    NCHUNK = PER_TILE // C2

    pltpu.async_copy(pred_hbm.at[pl.ds(base, C2)],
                     pbuf.at[pl.ds(0, C2)], sem_in)
    pltpu.async_copy(label_hbm.at[pl.ds(base, C2)],
                     lbuf.at[pl.ds(0, C2)], sem_in)

    plsc.subcore_barrier()

    def chunk(ci, _):
        b = lax.rem(ci, 3)
        boff = b * C2
        pltpu.make_async_copy(pred_hbm.at[pl.ds(0, C2)],
                              pbuf.at[pl.ds(boff, C2)], sem_in).wait()
        pltpu.make_async_copy(pred_hbm.at[pl.ds(0, C2)],
                              lbuf.at[pl.ds(boff, C2)], sem_in).wait()

        @pl.when(ci < NCHUNK - 1)
        def _():
            b2 = lax.rem(ci + 1, 3)
            off2 = base + (ci + 1) * C2
            pltpu.async_copy(pred_hbm.at[pl.ds(off2, C2)],
                             pbuf.at[pl.ds(b2 * C2, C2)], sem_in)
            pltpu.async_copy(label_hbm.at[pl.ds(off2, C2)],
                             lbuf.at[pl.ds(b2 * C2, C2)], sem_in)

        def vec_body(i, _2):
            for u in range(8):
                off = boff + i * 128 + u * 16
                p = pbuf[pl.ds(off, 16)]
                l = lbuf[pl.ds(off, 16)]
                e = 1.0 - p * (2.0 * l - 1.0)
                t = jnp.clip((hi_v - e) * iw_v, 0.0, K - 1.0)
                t = t + l * float(K)
                idxbuf[b * R2 + i, pl.ds(u * 16, 16)] = t.astype(jnp.int32)
            return 0

        lax.fori_loop(0, C2 // 128, vec_body, 0)

        @pl.when(ci > 0)
        def _():
            pltpu.make_async_copy(pred_hbm.at[pl.ds(0, C2)],
                                  zbuf, sem_sc).wait()

        def row_body(j, _2):
            irow = idxbuf.at[b * R2 + j]
            pltpu.async_copy(onesbuf, h2.at[irow], sem_sc, add=True)
            return 0

        lax.fori_loop(0, R2, row_body, 0)
        return 0

    lax.fori_loop(0, NCHUNK, chunk, 0)
    pltpu.make_async_copy(pred_hbm.at[pl.ds(0, C2)], zbuf, sem_sc).wait()
    plsc.subcore_barrier()

    pltpu.sync_copy(h2.at[pl.ds(sid * 2 * KS, 2 * KS)],
                    hist_hbm.at[pl.ds(cid * 2 * K + sid * 2 * KS, 2 * KS)])


_hist_call = pl.kernel(
    _hist_body,
    out_type=jax.ShapeDtypeStruct((NC * 2 * K,), jnp.float32),
    mesh=_mesh,
    scratch_types=[
        pltpu.VMEM((3 * C2,), jnp.float32),
        pltpu.VMEM((3 * C2,), jnp.float32),
        pltpu.VMEM((C2,), jnp.float32),
        pltpu.VMEM((3 * R2, 128), jnp.int32),
        pltpu.VMEM((128,), jnp.float32),
        pltpu.VMEM((2 * NW, 16), jnp.float32),
        pltpu.VMEM_SHARED((2 * K,), jnp.float32),
        pltpu.SemaphoreType.DMA,
        pltpu.SemaphoreType.DMA,
    ],
)


def _finish_body(mm_ref, h_ref, o_ref):
    m = h_ref[0, 0] + h_ref[1, 0]
    p = h_ref[0, 1] + h_ref[1, 1]
    lo = jnp.min(mm_ref[:, 0:16])
    hi = jnp.max(mm_ref[:, 16:32])
    rows = lax.broadcasted_iota(jnp.int32, (256, 256), 0)
    cols = lax.broadcasted_iota(jnp.int32, (256, 256), 1)
    ut = (rows <= cols).astype(jnp.float32)
    slt = (cols < rows).astype(jnp.float32)
    cp = jnp.dot(p, ut, preferred_element_type=jnp.float32,
                 precision=lax.Precision.HIGHEST)
    cn = jnp.dot(m, ut, preferred_element_type=jnp.float32,
                 precision=lax.Precision.HIGHEST)
    prow = jnp.sum(p, axis=1, keepdims=True)
    mrow = jnp.sum(m, axis=1, keepdims=True)
    cp = cp + jnp.dot(slt, prow, preferred_element_type=jnp.float32,
                      precision=lax.Precision.HIGHEST)
    cn = cn + jnp.dot(slt, mrow, preferred_element_type=jnp.float32,
                      precision=lax.Precision.HIGHEST)
    big_p = jnp.sum(p)
    cnt = cp + cn
    iou = jnp.where(cnt > 0.5,
                    1.0 - (big_p - cp) / jnp.maximum(big_p + cn, 1.0),
                    0.0)
    shift = (rows == cols - 1).astype(jnp.float32)
    prev_in_row = jnp.dot(iou, shift, preferred_element_type=jnp.float32,
                          precision=lax.Precision.HIGHEST)
    lastcol = iou[:, 255:256]
    shiftr = (cols == rows - 1).astype(jnp.float32)
    prev_row_last = jnp.dot(shiftr, lastcol, preferred_element_type=jnp.float32,
                            precision=lax.Precision.HIGHEST)
    col0 = (cols == 0).astype(jnp.float32)
    prev = prev_in_row + col0 * prev_row_last
    d = iou - prev
    w = jnp.maximum(hi - lo, 1e-30) / (K - 1.0)
    kk = (rows * 256 + cols).astype(jnp.float32)
    mid = jnp.maximum(hi - (kk + 0.5) * w, 0.0)
    loss = jnp.sum(mid * d)
    o_ref[...] = loss.reshape(1, 1)


_finish_call = pl.pallas_call(
    _finish_body,
    out_shape=jax.ShapeDtypeStruct((1, 1), jnp.float32),
    in_specs=[
        pl.BlockSpec((32, 32), lambda: (0, 0)),
        pl.BlockSpec((2, 2, 256, 256), lambda: (0, 0, 0, 0)),
    ],
    out_specs=pl.BlockSpec((1, 1), lambda: (0, 0)),
)


def kernel(prediction, label):
    pred = prediction.reshape(-1)
    lab = label.reshape(-1)
    mm = _minmax_call(pred, lab)
    hist = _hist_call(pred, lab, mm)
    out = _finish_call(mm.reshape(32, 32), hist.reshape(2, 2, 256, 256))
    return out.reshape(())
